# baseline (speedup 1.0000x reference)
import functools

import jax
import jax.numpy as jnp
from jax import lax
from jax.experimental import pallas as pl
from jax.experimental.pallas import tpu as pltpu
from jax.experimental.pallas import tpu_sc as plsc

NC = 2
NS = 16
NW = NC * NS
L = 16
CHUNK = 1024
DESC = 128
NDESC = CHUNK // DESC

f32 = jnp.float32
i32 = jnp.int32


def _mesh():
  return plsc.VectorSubcoreMesh(
      core_axis_name="c", subcore_axis_name="s", num_cores=NC,
      num_subcores=NS)


def _sc_params(tc_tiling=True):
  return pltpu.CompilerParams(needs_layout_passes=False,
                              use_tc_tiling_on_sc=tc_tiling,
                              disable_bounds_checks=True,
                              disable_semaphore_checks=True)


def _make_k1(n, n_chunks):
  epw = n_chunks * CHUNK
  assert n_chunks % 2 == 0

  @functools.partial(
      pl.kernel,
      out_type=(
          jax.ShapeDtypeStruct((NW, n), f32),
          jax.ShapeDtypeStruct((NW, n), f32),
          jax.ShapeDtypeStruct((NW, L), f32),
      ),
      mesh=_mesh(),
      compiler_params=_sc_params(),
      scratch_types=[
          pltpu.VMEM((n,), f32),
          pltpu.VMEM((n,), f32),
          pltpu.VMEM((CHUNK,), i32),
          pltpu.VMEM((CHUNK,), i32),
          pltpu.VMEM((CHUNK,), i32),
          pltpu.VMEM((CHUNK,), i32),
          pltpu.VMEM((CHUNK,), f32),
          pltpu.VMEM((CHUNK,), f32),
          pltpu.VMEM((L,), f32),
          pltpu.SemaphoreType.DMA,
          pltpu.SemaphoreType.DMA,
      ],
  )
  def k1(row_hbm, col_hbm, vals_hbm, diag_out, wrote_out, mmax_out,
         diag_t, wrote_t, rowv0, rowv1, colv0, colv1, valsv0, valsv1,
         mmaxv, sem0, sem1):
    cid = lax.axis_index("c")
    sid = lax.axis_index("s")
    wid = cid * NS + sid
    rowv = (rowv0, rowv1)
    colv = (colv0, colv1)
    valsv = (valsv0, valsv1)
    sem = (sem0, sem1)
    zero16 = jnp.zeros((L,), f32)
    ones16 = jnp.ones((L,), f32)

    @plsc.parallel_loop(0, n // L, unroll=8)
    def _zero(i):
      diag_t[pl.ds(i * L, L)] = zero16
      wrote_t[pl.ds(i * L, L)] = zero16

    ebase = wid * epw

    def fire_in(ci, b):
      base = pl.multiple_of(ebase + ci * CHUNK, CHUNK)
      pltpu.async_copy(row_hbm.at[pl.ds(base, CHUNK)], rowv[b], sem[b])
      pltpu.async_copy(col_hbm.at[pl.ds(base, CHUNK)], colv[b], sem[b])
      pltpu.async_copy(vals_hbm.at[pl.ds(base, CHUNK)], valsv[b], sem[b])

    def wait_in(b):
      pltpu.make_async_copy(
          row_hbm.at[pl.ds(0, CHUNK)], rowv[b], sem[b]).wait()
      pltpu.make_async_copy(
          col_hbm.at[pl.ds(0, CHUNK)], colv[b], sem[b]).wait()
      pltpu.make_async_copy(
          vals_hbm.at[pl.ds(0, CHUNK)], valsv[b], sem[b]).wait()

    fire_in(0, 0)

    def pair_body(cp, mmax_acc):
      for b in (0, 1):
        o = 1 - b
        ci = 2 * cp + b

        @pl.when(ci + 1 < n_chunks)
        def _():
          fire_in(ci + 1, o)

        wait_in(b)
        rv = rowv[b]
        cv = colv[b]
        vv = valsv[b]

        @plsc.parallel_loop(0, CHUNK // L, unroll=4, carry=mmax_acc)
        def acc_out(g, acc):
          r16 = rv[pl.ds(g * L, L)]
          c16 = cv[pl.ds(g * L, L)]
          v16 = vv[pl.ds(g * L, L)]
          m = r16 == c16
          acc = jnp.maximum(acc, jnp.abs(v16))
          plsc.store_scatter(diag_t, [r16], v16, mask=m)
          plsc.store_scatter(wrote_t, [r16], ones16, mask=m)
          return acc
        mmax_acc = acc_out
      return mmax_acc

    mmax_acc = lax.fori_loop(0, n_chunks // 2, pair_body,
                             jnp.zeros((L,), f32))
    mmaxv[...] = mmax_acc
    pltpu.sync_copy(diag_t, diag_out.at[wid])
    pltpu.sync_copy(wrote_t, wrote_out.at[wid])
    pltpu.sync_copy(mmaxv, mmax_out.at[wid])

  return k1


def _k2_body(diag_ref, wrote_ref, mmax_ref, b_ref, bb_ref, dd_ref, inv_ref):
  mmax = jnp.max(mmax_ref[...])
  b2d = b_ref[...]
  bmax = jnp.max(jnp.abs(b2d))
  d = jnp.zeros_like(b2d)
  for t in range(NW):
    d = jnp.where(wrote_ref[t] > 0.0, diag_ref[t], d)
  inv_m = 1.0 / mmax
  bb_ref[...] = b2d / bmax
  dd_ref[...] = d * inv_m
  inv_ref[...] = jnp.full((1, L), inv_m, f32)


def _run_k2(diag3, wrote3, mmax2, b2d, n):
  r = n // 128
  return pl.pallas_call(
      _k2_body,
      out_shape=(
          jax.ShapeDtypeStruct((r, 128), f32),
          jax.ShapeDtypeStruct((r, 128), f32),
          jax.ShapeDtypeStruct((1, L), f32),
      ),
  )(diag3, wrote3, mmax2, b2d)


def _make_k3(n, n_chunks, hidden):
  epw = n_chunks * CHUNK
  npt = n // NS
  assert n_chunks % 6 == 0 and npt == CHUNK

  @functools.partial(
      pl.kernel,
      out_type=(
          jax.ShapeDtypeStruct((NC, n, hidden), f32),
          jax.ShapeDtypeStruct((NC * n, hidden), f32),
      ),
      mesh=_mesh(),
      compiler_params=_sc_params(tc_tiling=False),
      scratch_types=[
          pltpu.VMEM((npt,), f32),
          pltpu.VMEM((npt,), f32),
          pltpu.VMEM((CHUNK, hidden), f32),
          pltpu.VMEM((CHUNK, hidden), f32),
          pltpu.VMEM((CHUNK,), i32),
          pltpu.VMEM((CHUNK,), i32),
          pltpu.VMEM((CHUNK,), i32),
          pltpu.VMEM((NDESC, DESC), i32),
          pltpu.VMEM((NDESC, DESC), i32),
          pltpu.VMEM((NDESC, DESC), i32),
          pltpu.VMEM((CHUNK,), f32),
          pltpu.VMEM((CHUNK,), f32),
          pltpu.VMEM((CHUNK,), f32),
          pltpu.VMEM((2, hidden), f32),
          pltpu.VMEM((hidden,), f32),
          pltpu.VMEM((L,), f32),
          pltpu.VMEM_SHARED((n, hidden), f32),
          pltpu.SemaphoreType.DMA,
          pltpu.SemaphoreType.DMA,
          pltpu.SemaphoreType.DMA,
          pltpu.SemaphoreType.DMA,
          pltpu.SemaphoreType.DMA,
          pltpu.SemaphoreType.DMA,
          pltpu.SemaphoreType.DMA,
      ],
  )
  def k3(row_hbm, col_hbm, vals_hbm, bb_hbm, dd_hbm, w1_hbm, b1_hbm,
         inv_hbm, agg_out, h_scr, bbv, ddv, gbuf0, gbuf1, colv0, colv1,
         colv2, rowv0, rowv1, rowv2, valsv0, valsv1, valsv2, w1v, b1v,
         invv, agg_sh, sem_in0, sem_in1, sem_in2, sem_g0, sem_g1,
         sem_s0, sem_s1):
    cid = lax.axis_index("c")
    sid = lax.axis_index("s")
    wid = cid * NS + sid
    nbase = sid * npt
    gbuf = (gbuf0, gbuf1)
    colv = (colv0, colv1, colv2)
    rowv = (rowv0, rowv1, rowv2)
    valsv = (valsv0, valsv1, valsv2)
    sem_in = (sem_in0, sem_in1, sem_in2)
    sem_g = (sem_g0, sem_g1)
    sem_s = (sem_s0, sem_s1)

    pltpu.sync_copy(w1_hbm, w1v)
    pltpu.sync_copy(b1_hbm, b1v)
    pltpu.sync_copy(inv_hbm, invv)
    pltpu.sync_copy(bb_hbm.at[pl.ds(pl.multiple_of(nbase, npt), npt)], bbv)
    pltpu.sync_copy(dd_hbm.at[pl.ds(pl.multiple_of(nbase, npt), npt)], ddv)

    w1a = w1v[0, pl.ds(0, L)]
    w1b = w1v[0, pl.ds(L, L)]
    w1c = w1v[1, pl.ds(0, L)]
    w1d = w1v[1, pl.ds(L, L)]
    b1a = b1v[pl.ds(0, L)]
    b1b = b1v[pl.ds(L, L)]
    iv = invv[...]
    zero16 = jnp.zeros((L,), f32)

    @plsc.parallel_loop(0, npt, unroll=4)
    def _h(q):
      qi = jnp.full((L,), q, i32)
      bbe = plsc.load_gather(bbv, [qi])
      dde = plsc.load_gather(ddv, [qi])
      h0 = jnp.maximum(bbe * w1a + dde * w1c + b1a, 0.0) * iv
      h1 = jnp.maximum(bbe * w1b + dde * w1d + b1b, 0.0) * iv
      gbuf0[q, pl.ds(0, L)] = h0
      gbuf0[q, pl.ds(L, L)] = h1
    pltpu.sync_copy(
        gbuf0, h_scr.at[pl.ds(pl.multiple_of(cid * n + nbase, npt), npt)])

    @plsc.parallel_loop(0, CHUNK, unroll=8)
    def _z(q):
      gbuf0[q, pl.ds(0, L)] = zero16
      gbuf0[q, pl.ds(L, L)] = zero16
    pltpu.sync_copy(gbuf0, agg_sh.at[pl.ds(nbase, npt)])

    plsc.subcore_barrier()

    ebase = wid * epw
    rbase = ebase // DESC

    def fire_in(ci, s):
      base = pl.multiple_of(ebase + ci * CHUNK, CHUNK)
      pltpu.async_copy(col_hbm.at[pl.ds(base, CHUNK)], colv[s], sem_in[s])
      pltpu.async_copy(vals_hbm.at[pl.ds(base, CHUNK)], valsv[s], sem_in[s])
      pltpu.async_copy(
          row_hbm.at[pl.ds(pl.multiple_of(rbase + ci * NDESC, NDESC), NDESC),
                     :], rowv[s], sem_in[s])

    def wait_in(s):
      pltpu.make_async_copy(
          col_hbm.at[pl.ds(0, CHUNK)], colv[s], sem_in[s]).wait()
      pltpu.make_async_copy(
          vals_hbm.at[pl.ds(0, CHUNK)], valsv[s], sem_in[s]).wait()
      pltpu.make_async_copy(
          row_hbm.at[pl.ds(0, NDESC), :], rowv[s], sem_in[s]).wait()

    coff = cid * n

    def off_add(s):
      cv = colv[s]

      @plsc.parallel_loop(0, CHUNK // L, unroll=4)
      def _off(g):
        cv[pl.ds(g * L, L)] = cv[pl.ds(g * L, L)] + coff

    def fire_gather(s, b):
      for j in range(NDESC):
        pltpu.async_copy(
            h_scr.at[colv[s].at[pl.ds(j * DESC, DESC)]],
            gbuf[b].at[pl.ds(j * DESC, DESC)], sem_g[b])

    def wait_gather(s, b):
      for j in range(NDESC):
        pltpu.make_async_copy(
            h_scr.at[colv[s].at[pl.ds(j * DESC, DESC)]],
            gbuf[b].at[pl.ds(j * DESC, DESC)], sem_g[b]).wait()

    def fire_scatter(b, s):
      for j in range(NDESC):
        pltpu.async_copy(
            gbuf[b].at[pl.ds(j * DESC, DESC)],
            agg_sh.at[rowv[s].at[j]], sem_s[b], add=True)

    def wait_scatter(b, s):
      for j in range(NDESC):
        pltpu.make_async_copy(
            gbuf[b].at[pl.ds(j * DESC, DESC)],
            agg_sh.at[rowv[s].at[j]], sem_s[b]).wait()

    def scale(b, s):
      gb = gbuf[b]
      vv = valsv[s]

      @plsc.parallel_loop(0, CHUNK, unroll=8)
      def _s(q):
        qi = jnp.full((L,), q, i32)
        val = plsc.load_gather(vv, [qi])
        gb[q, pl.ds(0, L)] = gb[q, pl.ds(0, L)] * val
        gb[q, pl.ds(L, L)] = gb[q, pl.ds(L, L)] * val

    fire_in(0, 0)
    wait_in(0)
    off_add(0)
    fire_gather(0, 0)
    fire_in(1, 1)

    def six_body(cp, carry):
      for k in range(6):
        b = k % 2
        o = 1 - b
        s = k % 3
        s1 = (k + 1) % 3
        s2 = (k + 2) % 3
        ci = 6 * cp + k

        @pl.when(ci + 1 < n_chunks)
        def _():
          wait_in(s1)
          off_add(s1)

        @pl.when(ci >= 1)
        def _():
          wait_scatter(o, s2)

        wait_gather(s, b)

        @pl.when(ci + 1 < n_chunks)
        def _():
          fire_gather(s1, o)

        @pl.when(ci + 2 < n_chunks)
        def _():
          fire_in(ci + 2, s2)

        scale(b, s)
        fire_scatter(b, s)
      return carry

    lax.fori_loop(0, n_chunks // 6, six_body, 0)
    wait_scatter((n_chunks - 1) % 2, (n_chunks - 1) % 3)

    plsc.subcore_barrier()
    pltpu.sync_copy(agg_sh.at[pl.ds(nbase, npt)],
                    agg_out.at[cid, pl.ds(nbase, npt)])

  return k3


def _k4_body(agg_ref, w2_ref, b2_ref, w3_ref, y_ref):
  agg = agg_ref[0] + agg_ref[1]
  h2 = jnp.maximum(
      jnp.dot(agg, w2_ref[...], preferred_element_type=f32) + b2_ref[...],
      0.0)
  y_ref[...] = jnp.sum(h2 * w3_ref[...], axis=1)


def _run_k4(agg_parts, w2, b2, w3, n, hidden):
  return pl.pallas_call(
      _k4_body,
      out_shape=jax.ShapeDtypeStruct((n,), f32),
  )(agg_parts, w2, b2.reshape(1, hidden), w3.reshape(1, hidden))


def _make_k5(n, n_chunks):
  epw = n_chunks * CHUNK
  npt = n // NS
  assert n_chunks % 6 == 0

  @functools.partial(
      pl.kernel,
      out_type=jax.ShapeDtypeStruct((NC, n), f32),
      mesh=_mesh(),
      compiler_params=_sc_params(tc_tiling=False),
      scratch_types=[
          pltpu.VMEM((n,), f32),
          pltpu.VMEM((CHUNK,), i32),
          pltpu.VMEM((CHUNK,), i32),
          pltpu.VMEM((CHUNK,), i32),
          pltpu.VMEM((NDESC, DESC), i32),
          pltpu.VMEM((NDESC, DESC), i32),
          pltpu.VMEM((NDESC, DESC), i32),
          pltpu.VMEM((CHUNK,), f32),
          pltpu.VMEM((CHUNK,), f32),
          pltpu.VMEM((CHUNK,), f32),
          pltpu.VMEM((CHUNK,), f32),
          pltpu.VMEM((CHUNK,), f32),
          pltpu.VMEM_SHARED((n,), f32),
          pltpu.SemaphoreType.DMA,
          pltpu.SemaphoreType.DMA,
          pltpu.SemaphoreType.DMA,
          pltpu.SemaphoreType.DMA,
          pltpu.SemaphoreType.DMA,
      ],
  )
  def k5(row_hbm, col_hbm, vals_hbm, y_hbm, p_out,
         ybuf, colv0, colv1, colv2, rowv0, rowv1, rowv2, valsv0, valsv1,
         valsv2, cbuf0, cbuf1, p_sh, sem_in0, sem_in1, sem_in2,
         sem_s0, sem_s1):
    cid = lax.axis_index("c")
    sid = lax.axis_index("s")
    wid = cid * NS + sid
    nbase = sid * npt
    colv = (colv0, colv1, colv2)
    rowv = (rowv0, rowv1, rowv2)
    valsv = (valsv0, valsv1, valsv2)
    cbuf = (cbuf0, cbuf1)
    sem_in = (sem_in0, sem_in1, sem_in2)
    sem_s = (sem_s0, sem_s1)
    zero16 = jnp.zeros((L,), f32)

    pltpu.sync_copy(y_hbm, ybuf)

    @plsc.parallel_loop(0, CHUNK // L, unroll=8)
    def _z(q):
      cbuf0[pl.ds(q * L, L)] = zero16
    pltpu.sync_copy(cbuf0.at[pl.ds(0, npt)], p_sh.at[pl.ds(nbase, npt)])

    plsc.subcore_barrier()

    ebase = wid * epw
    rbase = ebase // DESC

    def fire_in(ci, s):
      base = pl.multiple_of(ebase + ci * CHUNK, CHUNK)
      pltpu.async_copy(col_hbm.at[pl.ds(base, CHUNK)], colv[s], sem_in[s])
      pltpu.async_copy(vals_hbm.at[pl.ds(base, CHUNK)], valsv[s], sem_in[s])
      pltpu.async_copy(
          row_hbm.at[pl.ds(pl.multiple_of(rbase + ci * NDESC, NDESC), NDESC),
                     :], rowv[s], sem_in[s])

    def wait_in(s):
      pltpu.make_async_copy(
          col_hbm.at[pl.ds(0, CHUNK)], colv[s], sem_in[s]).wait()
      pltpu.make_async_copy(
          vals_hbm.at[pl.ds(0, CHUNK)], valsv[s], sem_in[s]).wait()
      pltpu.make_async_copy(
          row_hbm.at[pl.ds(0, NDESC), :], rowv[s], sem_in[s]).wait()

    def compute(s, b):
      cv = colv[s]
      vv = valsv[s]
      cb = cbuf[b]

      @plsc.parallel_loop(0, CHUNK // L, unroll=4)
      def _g(g):
        c16 = cv[pl.ds(g * L, L)]
        yv = plsc.load_gather(ybuf, [c16])
        cb[pl.ds(g * L, L)] = yv * vv[pl.ds(g * L, L)]

    def fire_scatter(b, s):
      for j in range(NDESC):
        pltpu.async_copy(
            cbuf[b].at[pl.ds(j * DESC, DESC)],
            p_sh.at[rowv[s].at[j]], sem_s[b], add=True)

    def wait_scatter(b, s):
      for j in range(NDESC):
        pltpu.make_async_copy(
            cbuf[b].at[pl.ds(j * DESC, DESC)],
            p_sh.at[rowv[s].at[j]], sem_s[b]).wait()

    fire_in(0, 0)
    fire_in(1, 1)

    def six_body(cp, carry):
      for k in range(6):
        b = k % 2
        o = 1 - b
        s = k % 3
        s2 = (k + 2) % 3
        ci = 6 * cp + k
        wait_in(s)
        compute(s, b)

        @pl.when(ci >= 1)
        def _():
          wait_scatter(o, s2)

        fire_scatter(b, s)

        @pl.when(ci + 2 < n_chunks)
        def _():
          fire_in(ci + 2, s2)
      return carry

    lax.fori_loop(0, n_chunks // 6, six_body, 0)
    wait_scatter((n_chunks - 1) % 2, (n_chunks - 1) % 3)

    plsc.subcore_barrier()
    pltpu.sync_copy(p_sh.at[pl.ds(nbase, npt)],
                    p_out.at[cid, pl.ds(nbase, npt)])

  return k5


def _k6_body(p_ref, b_ref, y_ref, out_ref):
  p = p_ref[0] + p_ref[1]
  b2d = b_ref[...]
  p_sq = jnp.sum(p * p)
  bp = jnp.sum(p * b2d)
  scaler = jnp.maximum(bp / p_sq, 1e-16)
  out_ref[...] = y_ref[...] * scaler


def _run_k6(p_parts3, b2d, y2d, n):
  r = n // 128
  return pl.pallas_call(
      _k6_body,
      out_shape=jax.ShapeDtypeStruct((r, 128), f32),
  )(p_parts3, b2d, y2d)


def kernel(b, m_indices, m_values, W1, b1, W2, b2, w3):
  n = b.shape[0]
  nnz = m_values.shape[0]
  hidden = W1.shape[1]

  per_worker = 6 * CHUNK * -(-nnz // (6 * CHUNK * NW))
  nnz_pad = per_worker * NW
  n_chunks = per_worker // CHUNK
  pad = nnz_pad - nnz

  it = jnp.arange(pad, dtype=i32)
  row = jnp.concatenate([m_indices[0], it % n])
  col = jnp.concatenate([m_indices[1], (it + 7) % n])
  vals = jnp.concatenate([m_values, jnp.zeros((pad,), f32)])
  row2d = row.reshape(-1, DESC)

  diag_c, wrote_c, mmax_c = _make_k1(n, n_chunks)(row, col, vals)

  r = n // 128
  bb2d, dd2d, inv2d = _run_k2(
      diag_c.reshape(NW, r, 128), wrote_c.reshape(NW, r, 128), mmax_c,
      b.reshape(r, 128), n)
  bb = bb2d.reshape(n)
  dd = dd2d.reshape(n)
  inv16 = inv2d.reshape(L)

  agg_parts, _ = _make_k3(n, n_chunks, hidden)(
      row2d, col, vals, bb, dd, W1, b1, inv16)

  y2 = _run_k4(agg_parts, W2, b2, w3, n, hidden)
  y = y2.reshape(n)

  p_parts = _make_k5(n, n_chunks)(row2d, col, vals, y)

  out = _run_k6(p_parts.reshape(NC, r, 128), b.reshape(r, 128),
                y.reshape(r, 128), n)
  return out.reshape(n)


# speedup vs baseline: 1.0007x; 1.0007x over previous
import functools

import jax
import jax.numpy as jnp
from jax import lax
from jax.experimental import pallas as pl
from jax.experimental.pallas import tpu as pltpu
from jax.experimental.pallas import tpu_sc as plsc

NC = 2
NS = 16
NW = NC * NS
L = 16
CHUNK = 1024
DESC = 128
NDESC = CHUNK // DESC

f32 = jnp.float32
i32 = jnp.int32


def _mesh():
  return plsc.VectorSubcoreMesh(
      core_axis_name="c", subcore_axis_name="s", num_cores=NC,
      num_subcores=NS)


def _sc_params(tc_tiling=True):
  return pltpu.CompilerParams(needs_layout_passes=False,
                              use_tc_tiling_on_sc=tc_tiling)


def _make_k1(n, n_chunks):
  epw = n_chunks * CHUNK
  assert n_chunks % 2 == 0

  @functools.partial(
      pl.kernel,
      out_type=(
          jax.ShapeDtypeStruct((NW, n), f32),
          jax.ShapeDtypeStruct((NW, n), f32),
          jax.ShapeDtypeStruct((NW, L), f32),
      ),
      mesh=_mesh(),
      compiler_params=_sc_params(),
      scratch_types=[
          pltpu.VMEM((n,), f32),
          pltpu.VMEM((n,), f32),
          pltpu.VMEM((CHUNK,), i32),
          pltpu.VMEM((CHUNK,), i32),
          pltpu.VMEM((CHUNK,), i32),
          pltpu.VMEM((CHUNK,), i32),
          pltpu.VMEM((CHUNK,), f32),
          pltpu.VMEM((CHUNK,), f32),
          pltpu.VMEM((L,), f32),
          pltpu.SemaphoreType.DMA,
          pltpu.SemaphoreType.DMA,
      ],
  )
  def k1(row_hbm, col_hbm, vals_hbm, diag_out, wrote_out, mmax_out,
         diag_t, wrote_t, rowv0, rowv1, colv0, colv1, valsv0, valsv1,
         mmaxv, sem0, sem1):
    cid = lax.axis_index("c")
    sid = lax.axis_index("s")
    wid = cid * NS + sid
    rowv = (rowv0, rowv1)
    colv = (colv0, colv1)
    valsv = (valsv0, valsv1)
    sem = (sem0, sem1)
    zero16 = jnp.zeros((L,), f32)
    ones16 = jnp.ones((L,), f32)

    @plsc.parallel_loop(0, n // L, unroll=8)
    def _zero(i):
      diag_t[pl.ds(i * L, L)] = zero16
      wrote_t[pl.ds(i * L, L)] = zero16

    ebase = wid * epw

    def fire_in(ci, b):
      base = pl.multiple_of(ebase + ci * CHUNK, CHUNK)
      pltpu.async_copy(row_hbm.at[pl.ds(base, CHUNK)], rowv[b], sem[b])
      pltpu.async_copy(col_hbm.at[pl.ds(base, CHUNK)], colv[b], sem[b])
      pltpu.async_copy(vals_hbm.at[pl.ds(base, CHUNK)], valsv[b], sem[b])

    def wait_in(b):
      pltpu.make_async_copy(
          row_hbm.at[pl.ds(0, CHUNK)], rowv[b], sem[b]).wait()
      pltpu.make_async_copy(
          col_hbm.at[pl.ds(0, CHUNK)], colv[b], sem[b]).wait()
      pltpu.make_async_copy(
          vals_hbm.at[pl.ds(0, CHUNK)], valsv[b], sem[b]).wait()

    fire_in(0, 0)

    def pair_body(cp, mmax_acc):
      for b in (0, 1):
        o = 1 - b
        ci = 2 * cp + b

        @pl.when(ci + 1 < n_chunks)
        def _():
          fire_in(ci + 1, o)

        wait_in(b)
        rv = rowv[b]
        cv = colv[b]
        vv = valsv[b]

        @plsc.parallel_loop(0, CHUNK // L, unroll=4, carry=mmax_acc)
        def acc_out(g, acc):
          r16 = rv[pl.ds(g * L, L)]
          c16 = cv[pl.ds(g * L, L)]
          v16 = vv[pl.ds(g * L, L)]
          m = r16 == c16
          acc = jnp.maximum(acc, jnp.abs(v16))
          plsc.store_scatter(diag_t, [r16], v16, mask=m)
          plsc.store_scatter(wrote_t, [r16], ones16, mask=m)
          return acc
        mmax_acc = acc_out
      return mmax_acc

    mmax_acc = lax.fori_loop(0, n_chunks // 2, pair_body,
                             jnp.zeros((L,), f32))
    mmaxv[...] = mmax_acc
    pltpu.sync_copy(diag_t, diag_out.at[wid])
    pltpu.sync_copy(wrote_t, wrote_out.at[wid])
    pltpu.sync_copy(mmaxv, mmax_out.at[wid])

  return k1


def _k2_body(diag_ref, wrote_ref, mmax_ref, b_ref, bb_ref, dd_ref, inv_ref):
  mmax = jnp.max(mmax_ref[...])
  b2d = b_ref[...]
  bmax = jnp.max(jnp.abs(b2d))
  d = jnp.zeros_like(b2d)
  for t in range(NW):
    d = jnp.where(wrote_ref[t] > 0.0, diag_ref[t], d)
  inv_m = 1.0 / mmax
  bb_ref[...] = b2d / bmax
  dd_ref[...] = d * inv_m
  inv_ref[...] = jnp.full((1, L), inv_m, f32)


def _run_k2(diag3, wrote3, mmax2, b2d, n):
  r = n // 128
  return pl.pallas_call(
      _k2_body,
      out_shape=(
          jax.ShapeDtypeStruct((r, 128), f32),
          jax.ShapeDtypeStruct((r, 128), f32),
          jax.ShapeDtypeStruct((1, L), f32),
      ),
  )(diag3, wrote3, mmax2, b2d)


def _make_k3(n, n_chunks, hidden):
  epw = n_chunks * CHUNK
  npt = n // NS
  assert n_chunks % 6 == 0 and npt == CHUNK

  @functools.partial(
      pl.kernel,
      out_type=(
          jax.ShapeDtypeStruct((NC, n, hidden), f32),
          jax.ShapeDtypeStruct((NC * n, hidden), f32),
      ),
      mesh=_mesh(),
      compiler_params=_sc_params(tc_tiling=False),
      scratch_types=[
          pltpu.VMEM((npt,), f32),
          pltpu.VMEM((npt,), f32),
          pltpu.VMEM((CHUNK, hidden), f32),
          pltpu.VMEM((CHUNK, hidden), f32),
          pltpu.VMEM((CHUNK,), i32),
          pltpu.VMEM((CHUNK,), i32),
          pltpu.VMEM((CHUNK,), i32),
          pltpu.VMEM((NDESC, DESC), i32),
          pltpu.VMEM((NDESC, DESC), i32),
          pltpu.VMEM((NDESC, DESC), i32),
          pltpu.VMEM((CHUNK,), f32),
          pltpu.VMEM((CHUNK,), f32),
          pltpu.VMEM((CHUNK,), f32),
          pltpu.VMEM((2, hidden), f32),
          pltpu.VMEM((hidden,), f32),
          pltpu.VMEM((L,), f32),
          pltpu.VMEM_SHARED((n, hidden), f32),
          pltpu.SemaphoreType.DMA,
          pltpu.SemaphoreType.DMA,
          pltpu.SemaphoreType.DMA,
          pltpu.SemaphoreType.DMA,
          pltpu.SemaphoreType.DMA,
          pltpu.SemaphoreType.DMA,
          pltpu.SemaphoreType.DMA,
      ],
  )
  def k3(row_hbm, col_hbm, vals_hbm, bb_hbm, dd_hbm, w1_hbm, b1_hbm,
         inv_hbm, agg_out, h_scr, bbv, ddv, gbuf0, gbuf1, colv0, colv1,
         colv2, rowv0, rowv1, rowv2, valsv0, valsv1, valsv2, w1v, b1v,
         invv, agg_sh, sem_in0, sem_in1, sem_in2, sem_g0, sem_g1,
         sem_s0, sem_s1):
    cid = lax.axis_index("c")
    sid = lax.axis_index("s")
    wid = cid * NS + sid
    nbase = sid * npt
    gbuf = (gbuf0, gbuf1)
    colv = (colv0, colv1, colv2)
    rowv = (rowv0, rowv1, rowv2)
    valsv = (valsv0, valsv1, valsv2)
    sem_in = (sem_in0, sem_in1, sem_in2)
    sem_g = (sem_g0, sem_g1)
    sem_s = (sem_s0, sem_s1)

    pltpu.sync_copy(w1_hbm, w1v)
    pltpu.sync_copy(b1_hbm, b1v)
    pltpu.sync_copy(inv_hbm, invv)
    pltpu.sync_copy(bb_hbm.at[pl.ds(pl.multiple_of(nbase, npt), npt)], bbv)
    pltpu.sync_copy(dd_hbm.at[pl.ds(pl.multiple_of(nbase, npt), npt)], ddv)

    w1a = w1v[0, pl.ds(0, L)]
    w1b = w1v[0, pl.ds(L, L)]
    w1c = w1v[1, pl.ds(0, L)]
    w1d = w1v[1, pl.ds(L, L)]
    b1a = b1v[pl.ds(0, L)]
    b1b = b1v[pl.ds(L, L)]
    iv = invv[...]
    zero16 = jnp.zeros((L,), f32)

    @plsc.parallel_loop(0, npt, unroll=4)
    def _h(q):
      qi = jnp.full((L,), q, i32)
      bbe = plsc.load_gather(bbv, [qi])
      dde = plsc.load_gather(ddv, [qi])
      h0 = jnp.maximum(bbe * w1a + dde * w1c + b1a, 0.0) * iv
      h1 = jnp.maximum(bbe * w1b + dde * w1d + b1b, 0.0) * iv
      gbuf0[q, pl.ds(0, L)] = h0
      gbuf0[q, pl.ds(L, L)] = h1
    pltpu.sync_copy(
        gbuf0, h_scr.at[pl.ds(pl.multiple_of(cid * n + nbase, npt), npt)])

    @plsc.parallel_loop(0, CHUNK, unroll=8)
    def _z(q):
      gbuf0[q, pl.ds(0, L)] = zero16
      gbuf0[q, pl.ds(L, L)] = zero16
    pltpu.sync_copy(gbuf0, agg_sh.at[pl.ds(nbase, npt)])

    plsc.subcore_barrier()

    ebase = wid * epw
    rbase = ebase // DESC

    def fire_in(ci, s):
      base = pl.multiple_of(ebase + ci * CHUNK, CHUNK)
      pltpu.async_copy(col_hbm.at[pl.ds(base, CHUNK)], colv[s], sem_in[s])
      pltpu.async_copy(vals_hbm.at[pl.ds(base, CHUNK)], valsv[s], sem_in[s])
      pltpu.async_copy(
          row_hbm.at[pl.ds(pl.multiple_of(rbase + ci * NDESC, NDESC), NDESC),
                     :], rowv[s], sem_in[s])

    def wait_in(s):
      pltpu.make_async_copy(
          col_hbm.at[pl.ds(0, CHUNK)], colv[s], sem_in[s]).wait()
      pltpu.make_async_copy(
          vals_hbm.at[pl.ds(0, CHUNK)], valsv[s], sem_in[s]).wait()
      pltpu.make_async_copy(
          row_hbm.at[pl.ds(0, NDESC), :], rowv[s], sem_in[s]).wait()

    coff = cid * n

    def off_add(s):
      cv = colv[s]

      @plsc.parallel_loop(0, CHUNK // L, unroll=4)
      def _off(g):
        cv[pl.ds(g * L, L)] = cv[pl.ds(g * L, L)] + coff

    def fire_gather(s, b):
      for j in range(NDESC):
        pltpu.async_copy(
            h_scr.at[colv[s].at[pl.ds(j * DESC, DESC)]],
            gbuf[b].at[pl.ds(j * DESC, DESC)], sem_g[b])

    def wait_gather(s, b):
      for j in range(NDESC):
        pltpu.make_async_copy(
            h_scr.at[colv[s].at[pl.ds(j * DESC, DESC)]],
            gbuf[b].at[pl.ds(j * DESC, DESC)], sem_g[b]).wait()

    def fire_scatter(b, s):
      for j in range(NDESC):
        pltpu.async_copy(
            gbuf[b].at[pl.ds(j * DESC, DESC)],
            agg_sh.at[rowv[s].at[j]], sem_s[b], add=True)

    def wait_scatter(b, s):
      for j in range(NDESC):
        pltpu.make_async_copy(
            gbuf[b].at[pl.ds(j * DESC, DESC)],
            agg_sh.at[rowv[s].at[j]], sem_s[b]).wait()

    def scale(b, s):
      gb = gbuf[b]
      vv = valsv[s]

      @plsc.parallel_loop(0, CHUNK, unroll=8)
      def _s(q):
        qi = jnp.full((L,), q, i32)
        val = plsc.load_gather(vv, [qi])
        gb[q, pl.ds(0, L)] = gb[q, pl.ds(0, L)] * val
        gb[q, pl.ds(L, L)] = gb[q, pl.ds(L, L)] * val

    fire_in(0, 0)
    wait_in(0)
    off_add(0)
    fire_gather(0, 0)
    fire_in(1, 1)

    def six_body(cp, carry):
      for k in range(6):
        b = k % 2
        o = 1 - b
        s = k % 3
        s1 = (k + 1) % 3
        s2 = (k + 2) % 3
        ci = 6 * cp + k

        @pl.when(ci + 1 < n_chunks)
        def _():
          wait_in(s1)
          off_add(s1)

        @pl.when(ci >= 1)
        def _():
          wait_scatter(o, s2)

        wait_gather(s, b)

        @pl.when(ci + 1 < n_chunks)
        def _():
          fire_gather(s1, o)

        @pl.when(ci + 2 < n_chunks)
        def _():
          fire_in(ci + 2, s2)

        scale(b, s)
        fire_scatter(b, s)
      return carry

    lax.fori_loop(0, n_chunks // 6, six_body, 0)
    wait_scatter((n_chunks - 1) % 2, (n_chunks - 1) % 3)

    plsc.subcore_barrier()
    pltpu.sync_copy(agg_sh.at[pl.ds(nbase, npt)],
                    agg_out.at[cid, pl.ds(nbase, npt)])

  return k3


def _k4_body(agg_ref, w2_ref, b2_ref, w3_ref, y_ref):
  agg = agg_ref[0] + agg_ref[1]
  h2 = jnp.maximum(
      jnp.dot(agg, w2_ref[...], preferred_element_type=f32) + b2_ref[...],
      0.0)
  y_ref[...] = jnp.sum(h2 * w3_ref[...], axis=1)


def _run_k4(agg_parts, w2, b2, w3, n, hidden):
  return pl.pallas_call(
      _k4_body,
      out_shape=jax.ShapeDtypeStruct((n,), f32),
  )(agg_parts, w2, b2.reshape(1, hidden), w3.reshape(1, hidden))


def _make_k5(n, n_chunks):
  epw = n_chunks * CHUNK
  npt = n // NS
  assert n_chunks % 6 == 0

  @functools.partial(
      pl.kernel,
      out_type=jax.ShapeDtypeStruct((NC, n), f32),
      mesh=_mesh(),
      compiler_params=_sc_params(tc_tiling=False),
      scratch_types=[
          pltpu.VMEM((n,), f32),
          pltpu.VMEM((CHUNK,), i32),
          pltpu.VMEM((CHUNK,), i32),
          pltpu.VMEM((CHUNK,), i32),
          pltpu.VMEM((NDESC, DESC), i32),
          pltpu.VMEM((NDESC, DESC), i32),
          pltpu.VMEM((NDESC, DESC), i32),
          pltpu.VMEM((CHUNK,), f32),
          pltpu.VMEM((CHUNK,), f32),
          pltpu.VMEM((CHUNK,), f32),
          pltpu.VMEM((CHUNK,), f32),
          pltpu.VMEM((CHUNK,), f32),
          pltpu.VMEM_SHARED((n,), f32),
          pltpu.SemaphoreType.DMA,
          pltpu.SemaphoreType.DMA,
          pltpu.SemaphoreType.DMA,
          pltpu.SemaphoreType.DMA,
          pltpu.SemaphoreType.DMA,
      ],
  )
  def k5(row_hbm, col_hbm, vals_hbm, y_hbm, p_out,
         ybuf, colv0, colv1, colv2, rowv0, rowv1, rowv2, valsv0, valsv1,
         valsv2, cbuf0, cbuf1, p_sh, sem_in0, sem_in1, sem_in2,
         sem_s0, sem_s1):
    cid = lax.axis_index("c")
    sid = lax.axis_index("s")
    wid = cid * NS + sid
    nbase = sid * npt
    colv = (colv0, colv1, colv2)
    rowv = (rowv0, rowv1, rowv2)
    valsv = (valsv0, valsv1, valsv2)
    cbuf = (cbuf0, cbuf1)
    sem_in = (sem_in0, sem_in1, sem_in2)
    sem_s = (sem_s0, sem_s1)
    zero16 = jnp.zeros((L,), f32)

    pltpu.sync_copy(y_hbm, ybuf)

    @plsc.parallel_loop(0, CHUNK // L, unroll=8)
    def _z(q):
      cbuf0[pl.ds(q * L, L)] = zero16
    pltpu.sync_copy(cbuf0.at[pl.ds(0, npt)], p_sh.at[pl.ds(nbase, npt)])

    plsc.subcore_barrier()

    ebase = wid * epw
    rbase = ebase // DESC

    def fire_in(ci, s):
      base = pl.multiple_of(ebase + ci * CHUNK, CHUNK)
      pltpu.async_copy(col_hbm.at[pl.ds(base, CHUNK)], colv[s], sem_in[s])
      pltpu.async_copy(vals_hbm.at[pl.ds(base, CHUNK)], valsv[s], sem_in[s])
      pltpu.async_copy(
          row_hbm.at[pl.ds(pl.multiple_of(rbase + ci * NDESC, NDESC), NDESC),
                     :], rowv[s], sem_in[s])

    def wait_in(s):
      pltpu.make_async_copy(
          col_hbm.at[pl.ds(0, CHUNK)], colv[s], sem_in[s]).wait()
      pltpu.make_async_copy(
          vals_hbm.at[pl.ds(0, CHUNK)], valsv[s], sem_in[s]).wait()
      pltpu.make_async_copy(
          row_hbm.at[pl.ds(0, NDESC), :], rowv[s], sem_in[s]).wait()

    def compute(s, b):
      cv = colv[s]
      vv = valsv[s]
      cb = cbuf[b]

      @plsc.parallel_loop(0, CHUNK // L, unroll=4)
      def _g(g):
        c16 = cv[pl.ds(g * L, L)]
        yv = plsc.load_gather(ybuf, [c16])
        cb[pl.ds(g * L, L)] = yv * vv[pl.ds(g * L, L)]

    def fire_scatter(b, s):
      for j in range(NDESC):
        pltpu.async_copy(
            cbuf[b].at[pl.ds(j * DESC, DESC)],
            p_sh.at[rowv[s].at[j]], sem_s[b], add=True)

    def wait_scatter(b, s):
      for j in range(NDESC):
        pltpu.make_async_copy(
            cbuf[b].at[pl.ds(j * DESC, DESC)],
            p_sh.at[rowv[s].at[j]], sem_s[b]).wait()

    fire_in(0, 0)
    fire_in(1, 1)

    def six_body(cp, carry):
      for k in range(6):
        b = k % 2
        o = 1 - b
        s = k % 3
        s2 = (k + 2) % 3
        ci = 6 * cp + k
        wait_in(s)
        compute(s, b)

        @pl.when(ci >= 1)
        def _():
          wait_scatter(o, s2)

        fire_scatter(b, s)

        @pl.when(ci + 2 < n_chunks)
        def _():
          fire_in(ci + 2, s2)
      return carry

    lax.fori_loop(0, n_chunks // 6, six_body, 0)
    wait_scatter((n_chunks - 1) % 2, (n_chunks - 1) % 3)

    plsc.subcore_barrier()
    pltpu.sync_copy(p_sh.at[pl.ds(nbase, npt)],
                    p_out.at[cid, pl.ds(nbase, npt)])

  return k5


def _k6_body(p_ref, b_ref, y_ref, out_ref):
  p = p_ref[0] + p_ref[1]
  b2d = b_ref[...]
  p_sq = jnp.sum(p * p)
  bp = jnp.sum(p * b2d)
  scaler = jnp.maximum(bp / p_sq, 1e-16)
  out_ref[...] = y_ref[...] * scaler


def _run_k6(p_parts3, b2d, y2d, n):
  r = n // 128
  return pl.pallas_call(
      _k6_body,
      out_shape=jax.ShapeDtypeStruct((r, 128), f32),
  )(p_parts3, b2d, y2d)


def kernel(b, m_indices, m_values, W1, b1, W2, b2, w3):
  n = b.shape[0]
  nnz = m_values.shape[0]
  hidden = W1.shape[1]

  per_worker = 6 * CHUNK * -(-nnz // (6 * CHUNK * NW))
  nnz_pad = per_worker * NW
  n_chunks = per_worker // CHUNK
  pad = nnz_pad - nnz

  it = jnp.arange(pad, dtype=i32)
  row = jnp.concatenate([m_indices[0], it % n])
  col = jnp.concatenate([m_indices[1], (it + 7) % n])
  vals = jnp.concatenate([m_values, jnp.zeros((pad,), f32)])
  row2d = row.reshape(-1, DESC)

  diag_c, wrote_c, mmax_c = _make_k1(n, n_chunks)(row, col, vals)

  r = n // 128
  bb2d, dd2d, inv2d = _run_k2(
      diag_c.reshape(NW, r, 128), wrote_c.reshape(NW, r, 128), mmax_c,
      b.reshape(r, 128), n)
  bb = bb2d.reshape(n)
  dd = dd2d.reshape(n)
  inv16 = inv2d.reshape(L)

  agg_parts, _ = _make_k3(n, n_chunks, hidden)(
      row2d, col, vals, bb, dd, W1, b1, inv16)

  y2 = _run_k4(agg_parts, W2, b2, w3, n, hidden)
  y = y2.reshape(n)

  p_parts = _make_k5(n, n_chunks)(row2d, col, vals, y)

  out = _run_k6(p_parts.reshape(NC, r, 128), b.reshape(r, 128),
                y.reshape(r, 128), n)
  return out.reshape(n)
